# baseline (speedup 1.0000x reference)
import functools
import math

import jax
import jax.numpy as jnp
from jax import lax
from jax.experimental import pallas as pl
from jax.experimental.pallas import tpu as pltpu
from jax.experimental.pallas import tpu_sc as plsc

EMBED_DIM = 64
SCALE = math.sqrt(EMBED_DIM)
LANES = 16
CHUNK = 128
K = 4


@functools.partial(jax.jit, static_argnames=("n_chunks_per_w", "num_cores"))
def _emb_lookup(tokens2d, table, *, n_chunks_per_w, num_cores):
    b_total = tokens2d.shape[0] * CHUNK
    ng = n_chunks_per_w // K
    assert ng % 2 == 0 and ng >= 4
    mesh = plsc.VectorSubcoreMesh(core_axis_name="c", subcore_axis_name="s")

    @functools.partial(
        pl.kernel,
        out_type=jax.ShapeDtypeStruct((b_total, EMBED_DIM), jnp.float32),
        mesh=mesh,
        compiler_params=pltpu.CompilerParams(use_tc_tiling_on_sc=False),
        scratch_types=[
            pltpu.VMEM((n_chunks_per_w, CHUNK), jnp.int32),
            pltpu.VMEM((2, K, CHUNK, EMBED_DIM), jnp.float32),
            pltpu.SemaphoreType.DMA,
            pltpu.SemaphoreType.DMA,
            pltpu.SemaphoreType.DMA,
            pltpu.SemaphoreType.DMA,
        ],
    )
    def k(tok_hbm, table_hbm, out_hbm, idx_v, rows_v, g0, g1, o0, o1):
        gsem = (g0, g1)
        osem = (o0, o1)
        wid = lax.axis_index("s") * num_cores + lax.axis_index("c")
        row0 = wid * n_chunks_per_w
        pltpu.sync_copy(tok_hbm.at[pl.ds(row0, n_chunks_per_w)], idx_v)

        def gather_desc(g, c, p):
            return pltpu.make_async_copy(
                table_hbm.at[idx_v.at[g * K + c]], rows_v.at[p, c], gsem[p]
            )

        def out_desc(g, c, p):
            j = g * K + c
            return pltpu.make_async_copy(
                rows_v.at[p, c], out_hbm.at[pl.ds((row0 + j) * CHUNK, CHUNK)], osem[p]
            )

        def scale_buf(p):
            def body(r, carry):
                for c in range(K):
                    for col in range(EMBED_DIM // LANES):
                        sl = pl.ds(col * LANES, LANES)
                        rows_v[p, c, r, sl] = rows_v[p, c, r, sl] * SCALE
                return carry

            lax.fori_loop(0, CHUNK, body, 0, unroll=4)

        def phase(g, p, fire_next, wait_prev_out):
            q = 1 - p
            if wait_prev_out:
                for c in range(K):
                    out_desc(g - 1, c, q).wait()
            if fire_next:
                for c in range(K):
                    gather_desc(g + 1, c, q).start()
            for c in range(K):
                gather_desc(g, c, p).wait()
            scale_buf(p)
            for c in range(K):
                out_desc(g, c, p).start()

        for c in range(K):
            gather_desc(0, c, 0).start()
        phase(0, 0, True, False)
        phase(1, 1, True, True)

        def group_body(g2, carry):
            phase(2 * g2, 0, True, True)
            phase(2 * g2 + 1, 1, True, True)
            return carry

        lax.fori_loop(1, ng // 2 - 1, group_body, 0)

        phase(ng - 2, 0, True, True)
        phase(ng - 1, 1, False, False)
        for c in range(K):
            out_desc(ng - 2, c, 0).wait()
            out_desc(ng - 1, c, 1).wait()

    return k(tokens2d, table)


def kernel(tokens, table):
    info = plsc.get_sparse_core_info()
    num_workers = info.num_cores * info.num_subcores
    batch, seq = tokens.shape
    b_total = batch * seq
    assert b_total % (num_workers * CHUNK * K) == 0
    n_chunks_per_w = b_total // (num_workers * CHUNK)
    tokens2d = tokens.reshape(b_total // CHUNK, CHUNK).astype(jnp.int32)
    out = _emb_lookup(
        tokens2d, table, n_chunks_per_w=n_chunks_per_w, num_cores=info.num_cores
    )
    return out.reshape(batch, seq, EMBED_DIM)


# speedup vs baseline: 1.2212x; 1.2212x over previous
import functools
import math

import jax
import jax.numpy as jnp
from jax import lax
from jax.experimental import pallas as pl
from jax.experimental.pallas import tpu as pltpu
from jax.experimental.pallas import tpu_sc as plsc

EMBED_DIM = 64
SCALE = math.sqrt(EMBED_DIM)
LANES = 16
CHUNK = 128
K = 2
PAD_DIM = 128


@functools.partial(jax.jit, static_argnames=("n_chunks_per_w", "num_cores"))
def _emb_lookup(tokens2d, table_p, *, n_chunks_per_w, num_cores):
    b_total = tokens2d.shape[0] * CHUNK
    ng = n_chunks_per_w // K
    assert ng % 2 == 0 and ng >= 6
    mesh = plsc.VectorSubcoreMesh(core_axis_name="c", subcore_axis_name="s")

    @functools.partial(
        pl.kernel,
        out_type=jax.ShapeDtypeStruct((b_total, PAD_DIM), jnp.float32),
        mesh=mesh,
        compiler_params=pltpu.CompilerParams(use_tc_tiling_on_sc=True),
        scratch_types=[
            pltpu.VMEM((n_chunks_per_w, CHUNK), jnp.int32),
            pltpu.VMEM((2, K, CHUNK, PAD_DIM), jnp.float32),
            pltpu.SemaphoreType.DMA,
            pltpu.SemaphoreType.DMA,
            pltpu.SemaphoreType.DMA,
            pltpu.SemaphoreType.DMA,
        ],
    )
    def k(tok_hbm, table_hbm, out_hbm, idx_v, rows_v, g0, g1, o0, o1):
        gsem = (g0, g1)
        osem = (o0, o1)
        wid = lax.axis_index("s") * num_cores + lax.axis_index("c")
        row0 = wid * n_chunks_per_w
        pltpu.sync_copy(tok_hbm.at[pl.ds(row0, n_chunks_per_w)], idx_v)

        def gather_desc(g, c, p):
            return pltpu.make_async_copy(
                table_hbm.at[idx_v.at[g * K + c]], rows_v.at[p, c], gsem[p]
            )

        def out_desc(g, c, p):
            j = g * K + c
            return pltpu.make_async_copy(
                rows_v.at[p, c],
                out_hbm.at[pl.ds((row0 + j) * CHUNK, CHUNK)],
                osem[p],
            )

        def scale_buf(p):
            def body(r, carry):
                for c in range(K):
                    for col in range(EMBED_DIM // LANES):
                        sl = pl.ds(col * LANES, LANES)
                        rows_v[p, c, r, sl] = rows_v[p, c, r, sl] * SCALE
                return carry

            lax.fori_loop(0, CHUNK, body, 0, unroll=4)

        def phase(g, p, fire_next, wait_prev_out):
            q = 1 - p
            if wait_prev_out:
                for c in range(K):
                    out_desc(g - 1, c, q).wait()
            if fire_next:
                for c in range(K):
                    gather_desc(g + 1, c, q).start()
            for c in range(K):
                gather_desc(g, c, p).wait()
            scale_buf(p)
            for c in range(K):
                out_desc(g, c, p).start()

        for c in range(K):
            gather_desc(0, c, 0).start()
        phase(0, 0, True, False)
        phase(1, 1, True, True)

        def group_body(g2, carry):
            phase(2 * g2, 0, True, True)
            phase(2 * g2 + 1, 1, True, True)
            return carry

        lax.fori_loop(1, ng // 2 - 1, group_body, 0)

        phase(ng - 2, 0, True, True)
        phase(ng - 1, 1, False, False)
        for c in range(K):
            out_desc(ng - 2, c, 0).wait()
            out_desc(ng - 1, c, 1).wait()

    return k(tokens2d, table_p)


def kernel(tokens, table):
    info = plsc.get_sparse_core_info()
    num_workers = info.num_cores * info.num_subcores
    batch, seq = tokens.shape
    b_total = batch * seq
    assert b_total % (num_workers * CHUNK * K) == 0
    n_chunks_per_w = b_total // (num_workers * CHUNK)
    tokens2d = tokens.reshape(b_total // CHUNK, CHUNK).astype(jnp.int32)
    table_p = jnp.pad(table, ((0, 0), (0, PAD_DIM - EMBED_DIM)))
    out = _emb_lookup(
        tokens2d, table_p, n_chunks_per_w=n_chunks_per_w, num_cores=info.num_cores
    )
    return out[:, :EMBED_DIM].reshape(batch, seq, EMBED_DIM)
